# Initial kernel scaffold; baseline (speedup 1.0000x reference)
import functools

import jax
import jax.numpy as jnp
from jax import lax
from jax.experimental import pallas as pl
from jax.experimental.pallas import tpu as pltpu
from jax.experimental.pallas import tpu_sc as plsc

MAXD = 4
L = 16
NC = 2
NS = 16
NW = NC * NS
CHUNK = 32768


def _sc_body(d_hbm, bias_hbm, out_hbm, table_v, din_v, dout_v):
    wid = lax.axis_index("s") * NC + lax.axis_index("c")
    n = d_hbm.shape[0]
    per_w = n // NW
    base = wid * per_w

    pltpu.sync_copy(bias_hbm, table_v)

    def chunk_body(ci, _):
        off = base + ci * CHUNK
        pltpu.sync_copy(d_hbm.at[pl.ds(off, CHUNK)], din_v)

        def vec_body(i, _):
            idx = din_v[pl.ds(i * L, L)]
            idx = jnp.minimum(jnp.maximum(idx, 0), MAXD)
            dout_v[pl.ds(i * L, L)] = plsc.load_gather(table_v, [idx])
            return 0

        lax.fori_loop(0, CHUNK // L, vec_body, 0, unroll=8)
        pltpu.sync_copy(dout_v, out_hbm.at[pl.ds(off, CHUNK)])
        return 0

    lax.fori_loop(0, per_w // CHUNK, chunk_body, 0)


def kernel(distances, distance_bias):
    shape = distances.shape
    n = distances.size
    d_flat = distances.reshape(n)
    bias16 = jnp.zeros((L,), jnp.float32).at[: distance_bias.shape[0]].set(
        distance_bias
    )

    mesh = plsc.VectorSubcoreMesh(core_axis_name="c", subcore_axis_name="s")
    out = pl.kernel(
        _sc_body,
        mesh=mesh,
        out_type=jax.ShapeDtypeStruct((n,), jnp.float32),
        scratch_types=[
            pltpu.VMEM((L,), jnp.float32),
            pltpu.VMEM((CHUNK,), jnp.int32),
            pltpu.VMEM((CHUNK,), jnp.float32),
        ],
    )(d_flat, bias16)
    return out.reshape(shape)


# speedup vs baseline: 319.6459x; 319.6459x over previous
import functools

import jax
import jax.numpy as jnp
from jax import lax
from jax.experimental import pallas as pl
from jax.experimental.pallas import tpu as pltpu
from jax.experimental.pallas import tpu_sc as plsc

MAXD = 4
L = 16
NC = 2
NS = 16
NW = NC * NS
CHUNK = 32768


def _sc_body(d_hbm, bias_hbm, out_hbm, table_v, din_v, dout_v):
    wid = lax.axis_index("s") * NC + lax.axis_index("c")
    n = d_hbm.shape[0]
    per_w = n // NW
    base = wid * per_w

    pltpu.sync_copy(bias_hbm, table_v)

    def chunk_body(ci, _):
        off = base + ci * CHUNK
        pltpu.sync_copy(d_hbm.at[pl.ds(off, CHUNK)], din_v)

        def vec_body(i, _):
            idx = din_v[pl.ds(i * L, L)]
            idx = jnp.minimum(jnp.maximum(idx, 0), MAXD)
            dout_v[pl.ds(i * L, L)] = plsc.load_gather(table_v, [idx])
            return 0

        lax.fori_loop(0, CHUNK // L, vec_body, 0, unroll=8)
        pltpu.sync_copy(dout_v, out_hbm.at[pl.ds(off, CHUNK)])
        return 0

    lax.fori_loop(0, per_w // CHUNK, chunk_body, 0)


def kernel(distances, distance_bias):
    shape = distances.shape
    n = distances.size
    d_flat = distances.reshape(n)
    bias16 = jnp.zeros((L,), jnp.float32).at[: distance_bias.shape[0]].set(
        distance_bias
    )

    mesh = plsc.VectorSubcoreMesh(core_axis_name="c", subcore_axis_name="s")
    out = pl.kernel(
        _sc_body,
        mesh=mesh,
        compiler_params=pltpu.CompilerParams(needs_layout_passes=False),
        out_type=jax.ShapeDtypeStruct((n,), jnp.float32),
        scratch_types=[
            pltpu.VMEM((L,), jnp.float32),
            pltpu.VMEM((CHUNK,), jnp.int32),
            pltpu.VMEM((CHUNK,), jnp.float32),
        ],
    )(d_flat, bias16)
    return out.reshape(shape)


# speedup vs baseline: 349.9235x; 1.0947x over previous
import functools

import jax
import jax.numpy as jnp
from jax import lax
from jax.experimental import pallas as pl
from jax.experimental.pallas import tpu as pltpu
from jax.experimental.pallas import tpu_sc as plsc

MAXD = 4
L = 16
NC = 2
NS = 16
NW = NC * NS
CHUNK = 16384
NBUF = 2


def _sc_body(d_hbm, bias_hbm, out_hbm, table_v, din_v, dout_v, sin, sout):
    wid = lax.axis_index("s") * NC + lax.axis_index("c")
    n = d_hbm.shape[0]
    per_w = n // NW
    nch = per_w // CHUNK
    base = wid * per_w

    pltpu.sync_copy(bias_hbm, table_v)

    def copy_in(c, b):
        return pltpu.async_copy(
            d_hbm.at[pl.ds(base + c * CHUNK, CHUNK)], din_v[b], sin[b]
        )

    for c in range(min(NBUF, nch)):
        copy_in(c, c % NBUF)

    out_copies = {}
    for c in range(nch):
        b = c % NBUF
        pltpu.make_async_copy(
            d_hbm.at[pl.ds(base + c * CHUNK, CHUNK)], din_v[b], sin[b]
        ).wait()
        if c >= NBUF:
            out_copies.pop(c - NBUF).wait()

        def vec_body(i, _, b=b):
            idx = din_v[b][pl.ds(i * L, L)]
            idx = jnp.minimum(jnp.maximum(idx, 0), MAXD)
            dout_v[b][pl.ds(i * L, L)] = plsc.load_gather(table_v, [idx])
            return 0

        lax.fori_loop(0, CHUNK // L, vec_body, 0, unroll=8)
        out_copies[c] = pltpu.async_copy(
            dout_v[b], out_hbm.at[pl.ds(base + c * CHUNK, CHUNK)], sout[b]
        )
        if c + NBUF < nch:
            copy_in(c + NBUF, b)
    for c in sorted(out_copies):
        out_copies.pop(c).wait()


def kernel(distances, distance_bias):
    shape = distances.shape
    n = distances.size
    d_flat = distances.reshape(n)
    bias16 = jnp.zeros((L,), jnp.float32).at[: distance_bias.shape[0]].set(
        distance_bias
    )

    mesh = plsc.VectorSubcoreMesh(core_axis_name="c", subcore_axis_name="s")
    out = pl.kernel(
        _sc_body,
        mesh=mesh,
        compiler_params=pltpu.CompilerParams(needs_layout_passes=False),
        out_type=jax.ShapeDtypeStruct((n,), jnp.float32),
        scratch_types=[
            pltpu.VMEM((L,), jnp.float32),
            [pltpu.VMEM((CHUNK,), jnp.int32) for _ in range(NBUF)],
            [pltpu.VMEM((CHUNK,), jnp.float32) for _ in range(NBUF)],
            [pltpu.SemaphoreType.DMA for _ in range(NBUF)],
            [pltpu.SemaphoreType.DMA for _ in range(NBUF)],
        ],
    )(d_flat, bias16)
    return out.reshape(shape)


# speedup vs baseline: 865.6987x; 2.4740x over previous
import functools

import jax
import jax.numpy as jnp
from jax import lax
from jax.experimental import pallas as pl
from jax.experimental.pallas import tpu as pltpu
from jax.experimental.pallas import tpu_sc as plsc

MAXD = 4
L = 16
NC = 2
NS = 16
NW = NC * NS
CHUNK = 16384
NBUF = 2


def _sc_body(d_hbm, bias_hbm, out_hbm, table_v, din_v, dout_v, sin, sout):
    wid = lax.axis_index("s") * NC + lax.axis_index("c")
    n = d_hbm.shape[0]
    per_w = n // NW
    nch = per_w // CHUNK
    base = wid * per_w

    pltpu.sync_copy(bias_hbm, table_v)

    def copy_in(c, b):
        return pltpu.async_copy(
            d_hbm.at[pl.ds(base + c * CHUNK, CHUNK)], din_v[b], sin[b]
        )

    for c in range(min(NBUF, nch)):
        copy_in(c, c % NBUF)

    out_copies = {}
    for c in range(nch):
        b = c % NBUF
        pltpu.make_async_copy(
            d_hbm.at[pl.ds(base + c * CHUNK, CHUNK)], din_v[b], sin[b]
        ).wait()
        if c >= NBUF:
            out_copies.pop(c - NBUF).wait()

        @plsc.parallel_loop(0, CHUNK, step=L, unroll=8)
        def vec_body(i, b=b):
            idx = din_v[b][pl.ds(i, L)]
            idx = jnp.minimum(jnp.maximum(idx, 0), MAXD)
            dout_v[b][pl.ds(i, L)] = plsc.load_gather(table_v, [idx])
        out_copies[c] = pltpu.async_copy(
            dout_v[b], out_hbm.at[pl.ds(base + c * CHUNK, CHUNK)], sout[b]
        )
        if c + NBUF < nch:
            copy_in(c + NBUF, b)
    for c in sorted(out_copies):
        out_copies.pop(c).wait()


def kernel(distances, distance_bias):
    shape = distances.shape
    n = distances.size
    d_flat = distances.reshape(n)
    bias16 = jnp.zeros((L,), jnp.float32).at[: distance_bias.shape[0]].set(
        distance_bias
    )

    mesh = plsc.VectorSubcoreMesh(core_axis_name="c", subcore_axis_name="s")
    out = pl.kernel(
        _sc_body,
        mesh=mesh,
        compiler_params=pltpu.CompilerParams(needs_layout_passes=False),
        out_type=jax.ShapeDtypeStruct((n,), jnp.float32),
        scratch_types=[
            pltpu.VMEM((L,), jnp.float32),
            [pltpu.VMEM((CHUNK,), jnp.int32) for _ in range(NBUF)],
            [pltpu.VMEM((CHUNK,), jnp.float32) for _ in range(NBUF)],
            [pltpu.SemaphoreType.DMA for _ in range(NBUF)],
            [pltpu.SemaphoreType.DMA for _ in range(NBUF)],
        ],
    )(d_flat, bias16)
    return out.reshape(shape)
